# baseline (speedup 1.0000x reference)
import functools

import jax
import jax.numpy as jnp
from jax import lax
from jax.experimental import pallas as pl
from jax.experimental.pallas import tpu as pltpu
from jax.experimental.pallas import tpu_sc as plsc

_VOCAB = 100000
_EMBED = 128
_BATCH = 4096
_MAXN = 1.0

_NC = 2
_NS = 16
_NW = _NC * _NS
_BPW = _BATCH // _NW

_BV = 512
_NV = (_VOCAB + _BV - 1) // _BV


@functools.partial(
    pl.kernel,
    mesh=plsc.VectorSubcoreMesh(core_axis_name="c", subcore_axis_name="s"),
    out_type=jax.ShapeDtypeStruct((_BATCH, _EMBED), jnp.float32),
    scratch_types=[
        pltpu.VMEM((_BPW,), jnp.int32),
        pltpu.VMEM((_BPW, _EMBED), jnp.float32),
        pltpu.SemaphoreType.DMA,
    ],
)
def _sc_gather(table_hbm, idx_hbm, out_hbm, idx_v, rows_v, sem):
    wid = lax.axis_index("s") * _NC + lax.axis_index("c")
    base = wid * _BPW
    pltpu.sync_copy(idx_hbm.at[pl.ds(base, _BPW)], idx_v)
    pltpu.async_copy(table_hbm.at[idx_v], rows_v, sem).wait()
    pltpu.sync_copy(rows_v, out_hbm.at[pl.ds(base, _BPW)])


def _proj_body(e_ref, w_ref, b_ref, o_ref):
    g = e_ref[...]
    ss = jnp.sum(g * g, axis=1, keepdims=True)
    norm = jnp.sqrt(ss)
    scale = jnp.minimum(1.0, _MAXN / jnp.maximum(norm, 1e-7))
    e = g * scale
    o_ref[...] = (
        lax.dot_general(
            e, w_ref[...], (((1,), (1,)), ((), ())),
            preferred_element_type=jnp.float32,
        )
        + b_ref[...]
    )


def kernel(x, emb_table, W, b):
    e_raw = _sc_gather(emb_table, x.astype(jnp.int32))
    b2 = b.reshape(1, _VOCAB)
    out = pl.pallas_call(
        _proj_body,
        grid=(_NV,),
        in_specs=[
            pl.BlockSpec((_BATCH, _EMBED), lambda j: (0, 0)),
            pl.BlockSpec((_BV, _EMBED), lambda j: (j, 0)),
            pl.BlockSpec((1, _BV), lambda j: (0, j)),
        ],
        out_specs=pl.BlockSpec((_BATCH, _BV), lambda j: (0, j)),
        out_shape=jax.ShapeDtypeStruct((_BATCH, _VOCAB), jnp.float32),
        compiler_params=pltpu.CompilerParams(
            dimension_semantics=("arbitrary",)
        ),
    )(e_raw, W, b2)
    return out


# speedup vs baseline: 3.2543x; 3.2543x over previous
import functools

import jax
import jax.numpy as jnp
from jax import lax
from jax.experimental import pallas as pl
from jax.experimental.pallas import tpu as pltpu
from jax.experimental.pallas import tpu_sc as plsc

_VOCAB = 100000
_EMBED = 128
_BATCH = 4096
_MAXN = 1.0

_NC = 2
_NS = 16
_NW = _NC * _NS
_BPW = _BATCH // _NW

_BV = 512
_NV = (_VOCAB + _BV - 1) // _BV


@functools.partial(
    pl.kernel,
    mesh=plsc.VectorSubcoreMesh(core_axis_name="c", subcore_axis_name="s"),
    out_type=jax.ShapeDtypeStruct((_BATCH, _EMBED), jnp.float32),
    scratch_types=[
        pltpu.VMEM((_BPW,), jnp.int32),
        pltpu.VMEM((_BPW, _EMBED), jnp.float32),
        pltpu.SemaphoreType.DMA,
    ],
)
def _sc_gather(table_hbm, idx_hbm, out_hbm, idx_v, rows_v, sem):
    wid = lax.axis_index("s") * _NC + lax.axis_index("c")
    base = wid * _BPW
    pltpu.sync_copy(idx_hbm.at[pl.ds(base, _BPW)], idx_v)
    pltpu.async_copy(table_hbm.at[idx_v], rows_v, sem).wait()
    pltpu.sync_copy(rows_v, out_hbm.at[pl.ds(base, _BPW)])


def _proj_body(e_ref, w_ref, b_ref, o_ref):
    g = e_ref[...]
    ss = jnp.sum(g * g, axis=1, keepdims=True)
    norm = jnp.sqrt(ss)
    scale = jnp.minimum(1.0, _MAXN / jnp.maximum(norm, 1e-7))
    e = g * scale
    o_ref[...] = (
        lax.dot_general(
            w_ref[...], e, (((1,), (1,)), ((), ())),
            preferred_element_type=jnp.float32,
        )
        + b_ref[...]
    )


def kernel(x, emb_table, W, b):
    e_raw = _sc_gather(emb_table, x.astype(jnp.int32))
    b2 = b.reshape(_VOCAB, 1)
    out_t = pl.pallas_call(
        _proj_body,
        grid=(_NV,),
        in_specs=[
            pl.BlockSpec((_BATCH, _EMBED), lambda j: (0, 0)),
            pl.BlockSpec((_BV, _EMBED), lambda j: (j, 0)),
            pl.BlockSpec((_BV, 1), lambda j: (j, 0)),
        ],
        out_specs=pl.BlockSpec((_BV, _BATCH), lambda j: (j, 0)),
        out_shape=jax.ShapeDtypeStruct((_VOCAB, _BATCH), jnp.float32),
        compiler_params=pltpu.CompilerParams(
            dimension_semantics=("arbitrary",)
        ),
    )(e_raw, W, b2)
    return out_t.T


# speedup vs baseline: 3.6853x; 1.1325x over previous
import functools

import jax
import jax.numpy as jnp
from jax import lax
from jax.experimental import pallas as pl
from jax.experimental.pallas import tpu as pltpu
from jax.experimental.pallas import tpu_sc as plsc

_VOCAB = 100000
_EMBED = 128
_BATCH = 4096
_MAXN = 1.0

_NC = 2
_NS = 16
_NW = _NC * _NS
_BPW = _BATCH // _NW

_BV = 1024
_NV = (_VOCAB + _BV - 1) // _BV


@functools.partial(
    pl.kernel,
    mesh=plsc.VectorSubcoreMesh(core_axis_name="c", subcore_axis_name="s"),
    out_type=jax.ShapeDtypeStruct((_BATCH, _EMBED), jnp.float32),
    scratch_types=[
        pltpu.VMEM((_BPW,), jnp.int32),
        pltpu.VMEM((_BPW, _EMBED), jnp.float32),
        pltpu.SemaphoreType.DMA,
    ],
)
def _sc_gather(table_hbm, idx_hbm, out_hbm, idx_v, rows_v, sem):
    wid = lax.axis_index("s") * _NC + lax.axis_index("c")
    base = wid * _BPW
    pltpu.sync_copy(idx_hbm.at[pl.ds(base, _BPW)], idx_v)
    pltpu.async_copy(table_hbm.at[idx_v], rows_v, sem).wait()
    pltpu.sync_copy(rows_v, out_hbm.at[pl.ds(base, _BPW)])


def _proj_body(e_ref, w_ref, b_ref, o_ref, e_scr):
    @pl.when(pl.program_id(0) == 0)
    def _():
        g = e_ref[...]
        ss = jnp.sum(g * g, axis=1, keepdims=True)
        norm = jnp.sqrt(ss)
        scale = jnp.minimum(1.0, _MAXN / jnp.maximum(norm, 1e-7))
        e_scr[...] = g * scale

    o_ref[...] = (
        lax.dot_general(
            w_ref[...], e_scr[...], (((1,), (1,)), ((), ())),
            preferred_element_type=jnp.float32,
        )
        + jnp.transpose(b_ref[...])
    )


def kernel(x, emb_table, W, b):
    e_raw = _sc_gather(emb_table, x.astype(jnp.int32))
    b2 = b.reshape(1, _VOCAB)
    out_t = pl.pallas_call(
        _proj_body,
        grid=(_NV,),
        in_specs=[
            pl.BlockSpec((_BATCH, _EMBED), lambda j: (0, 0)),
            pl.BlockSpec((_BV, _EMBED), lambda j: (j, 0)),
            pl.BlockSpec((1, _BV), lambda j: (0, j)),
        ],
        out_specs=pl.BlockSpec((_BV, _BATCH), lambda j: (j, 0)),
        out_shape=jax.ShapeDtypeStruct((_VOCAB, _BATCH), jnp.float32),
        scratch_shapes=[pltpu.VMEM((_BATCH, _EMBED), jnp.float32)],
        compiler_params=pltpu.CompilerParams(
            dimension_semantics=("arbitrary",)
        ),
    )(e_raw, W, b2)
    return out_t.T


# speedup vs baseline: 3.6855x; 1.0000x over previous
import functools

import jax
import jax.numpy as jnp
from jax import lax
from jax.experimental import pallas as pl
from jax.experimental.pallas import tpu as pltpu
from jax.experimental.pallas import tpu_sc as plsc

_VOCAB = 100000
_EMBED = 128
_BATCH = 4096
_MAXN = 1.0

_NC = 2
_NS = 16
_NW = _NC * _NS
_BPW = _BATCH // _NW

_BV = 1280
_NV = (_VOCAB + _BV - 1) // _BV


@functools.partial(
    pl.kernel,
    mesh=plsc.VectorSubcoreMesh(core_axis_name="c", subcore_axis_name="s"),
    out_type=jax.ShapeDtypeStruct((_BATCH, _EMBED), jnp.float32),
    scratch_types=[
        pltpu.VMEM((_BPW,), jnp.int32),
        pltpu.VMEM((_BPW, _EMBED), jnp.float32),
        pltpu.SemaphoreType.DMA,
    ],
)
def _sc_gather(table_hbm, idx_hbm, out_hbm, idx_v, rows_v, sem):
    wid = lax.axis_index("s") * _NC + lax.axis_index("c")
    base = wid * _BPW
    pltpu.sync_copy(idx_hbm.at[pl.ds(base, _BPW)], idx_v)
    pltpu.async_copy(table_hbm.at[idx_v], rows_v, sem).wait()
    pltpu.sync_copy(rows_v, out_hbm.at[pl.ds(base, _BPW)])


def _proj_body(e_ref, w_ref, b_ref, o_ref, e_scr):
    @pl.when(pl.program_id(0) == 0)
    def _():
        g = e_ref[...]
        ss = jnp.sum(g * g, axis=1, keepdims=True)
        norm = jnp.sqrt(ss)
        scale = jnp.minimum(1.0, _MAXN / jnp.maximum(norm, 1e-7))
        e_scr[...] = g * scale

    o_ref[...] = (
        lax.dot_general(
            w_ref[...], e_scr[...], (((1,), (1,)), ((), ())),
            preferred_element_type=jnp.float32,
        )
        + jnp.transpose(b_ref[...])
    )


def kernel(x, emb_table, W, b):
    e_raw = _sc_gather(emb_table, x.astype(jnp.int32))
    b2 = b.reshape(1, _VOCAB)
    out_t = pl.pallas_call(
        _proj_body,
        grid=(_NV,),
        in_specs=[
            pl.BlockSpec((_BATCH, _EMBED), lambda j: (0, 0)),
            pl.BlockSpec((_BV, _EMBED), lambda j: (j, 0)),
            pl.BlockSpec((1, _BV), lambda j: (0, j)),
        ],
        out_specs=pl.BlockSpec((_BV, _BATCH), lambda j: (j, 0)),
        out_shape=jax.ShapeDtypeStruct((_VOCAB, _BATCH), jnp.float32),
        scratch_shapes=[pltpu.VMEM((_BATCH, _EMBED), jnp.float32)],
        compiler_params=pltpu.CompilerParams(
            dimension_semantics=("arbitrary",)
        ),
    )(e_raw, W, b2)
    return out_t.T
